# baseline (speedup 1.0000x reference)
import jax
import jax.numpy as jnp
from jax import lax
from jax.experimental import pallas as pl
from jax.experimental.pallas import tpu as pltpu

_GRID = 8


def _body(x_ref, s_ref, c_ref, t_ref, cond_ref, w_ref, wc_ref, of_ref, oc_ref):
    d = w_ref.shape[0]
    rep = 128 // d
    w = w_ref[...]
    wrow = jnp.concatenate([w] * rep, axis=1)
    wbig = jnp.concatenate([wrow] * rep, axis=0)
    ii = lax.broadcasted_iota(jnp.int32, (128, 128), 0) // d
    jj = lax.broadcasted_iota(jnp.int32, (128, 128), 1) // d
    wbig = jnp.where(ii == jj, wbig, 0.0)

    cv = jnp.dot(jnp.mean(cond_ref[...], axis=0, keepdims=True), wc_ref[...],
                 preferred_element_type=jnp.float32)
    cvp = jnp.concatenate([cv] * rep, axis=1)

    tt = t_ref[0, 0]
    xw = jnp.dot(x_ref[...], wbig, preferred_element_type=jnp.float32)
    sw = jnp.dot(s_ref[...], wbig, preferred_element_type=jnp.float32)
    of_ref[...] = xw * tt + cvp + jnp.tanh(sw)
    oc_ref[...] = c_ref[...]


def kernel(x_t_feats, x_t_coords, tex_feats, tex_coords, shape_feats,
           shape_coords, t, cond, coords_len_list, W, Wc):
    T, D = x_t_feats.shape
    dc = x_t_coords.shape[1]
    B, DCOND = cond.shape
    rep = 128 // D
    Rf = T // rep
    Rc = T * dc // 128

    xp = x_t_feats.reshape(Rf, 128)
    sp = shape_feats.reshape(Rf, 128)
    cp = x_t_coords.reshape(Rc, 128)
    t2 = t.reshape(1, 1)

    fblk = Rf // _GRID
    cblk = Rc // _GRID

    ofp, ocp = pl.pallas_call(
        _body,
        grid=(_GRID,),
        in_specs=[
            pl.BlockSpec((fblk, 128), lambda i: (i, 0)),
            pl.BlockSpec((fblk, 128), lambda i: (i, 0)),
            pl.BlockSpec((cblk, 128), lambda i: (i, 0)),
            pl.BlockSpec((1, 1), lambda i: (0, 0)),
            pl.BlockSpec((B, DCOND), lambda i: (0, 0)),
            pl.BlockSpec((D, D), lambda i: (0, 0)),
            pl.BlockSpec((DCOND, D), lambda i: (0, 0)),
        ],
        out_specs=[
            pl.BlockSpec((fblk, 128), lambda i: (i, 0)),
            pl.BlockSpec((cblk, 128), lambda i: (i, 0)),
        ],
        out_shape=[
            jax.ShapeDtypeStruct((Rf, 128), jnp.float32),
            jax.ShapeDtypeStruct((Rc, 128), jnp.int32),
        ],
        compiler_params=pltpu.CompilerParams(
            dimension_semantics=("arbitrary",)),
    )(xp, sp, cp, t2, cond, W, Wc)

    return ofp.reshape(T, D), ocp.reshape(T, dc)


# speedup vs baseline: 1.5923x; 1.5923x over previous
import jax
import jax.numpy as jnp
from jax.experimental import pallas as pl
from jax.experimental.pallas import tpu as pltpu

_GRID = 8


def _body(x_ref, s_ref, c_ref, t_ref, cond_ref, w_ref, wc_ref, of_ref, oc_ref):
    cv = jnp.dot(jnp.mean(cond_ref[...], axis=0, keepdims=True), wc_ref[...],
                 preferred_element_type=jnp.float32)
    tt = t_ref[0, 0]
    xw = jnp.dot(x_ref[...], w_ref[...], preferred_element_type=jnp.float32)
    sw = jnp.dot(s_ref[...], w_ref[...], preferred_element_type=jnp.float32)
    of_ref[...] = xw * tt + cv + jnp.tanh(sw)
    oc_ref[...] = c_ref[...]


def kernel(x_t_feats, x_t_coords, tex_feats, tex_coords, shape_feats,
           shape_coords, t, cond, coords_len_list, W, Wc):
    T, D = x_t_feats.shape
    dc = x_t_coords.shape[1]
    B, DCOND = cond.shape
    blk = T // _GRID
    t2 = t.reshape(1, 1)

    of, oc = pl.pallas_call(
        _body,
        grid=(_GRID,),
        in_specs=[
            pl.BlockSpec((blk, D), lambda i: (i, 0)),
            pl.BlockSpec((blk, D), lambda i: (i, 0)),
            pl.BlockSpec((blk, dc), lambda i: (i, 0)),
            pl.BlockSpec((1, 1), lambda i: (0, 0)),
            pl.BlockSpec((B, DCOND), lambda i: (0, 0)),
            pl.BlockSpec((D, D), lambda i: (0, 0)),
            pl.BlockSpec((DCOND, D), lambda i: (0, 0)),
        ],
        out_specs=[
            pl.BlockSpec((blk, D), lambda i: (i, 0)),
            pl.BlockSpec((blk, dc), lambda i: (i, 0)),
        ],
        out_shape=[
            jax.ShapeDtypeStruct((T, D), jnp.float32),
            jax.ShapeDtypeStruct((T, dc), jnp.int32),
        ],
        compiler_params=pltpu.CompilerParams(
            dimension_semantics=("arbitrary",)),
    )(x_t_feats, shape_feats, x_t_coords, t2, cond, W, Wc)

    return of, oc


# speedup vs baseline: 2.4425x; 1.5339x over previous
import jax
import jax.numpy as jnp
from jax.experimental import pallas as pl
from jax.experimental.pallas import tpu as pltpu

_GRID = 8


def _body(x_ref, s_ref, t_ref, cond_ref, w_ref, wc_ref, of_ref):
    cv = jnp.dot(jnp.mean(cond_ref[...], axis=0, keepdims=True), wc_ref[...],
                 preferred_element_type=jnp.float32)
    tt = t_ref[0, 0]
    xw = jnp.dot(x_ref[...], w_ref[...], preferred_element_type=jnp.float32)
    sw = jnp.dot(s_ref[...], w_ref[...], preferred_element_type=jnp.float32)
    of_ref[...] = xw * tt + cv + jnp.tanh(sw)


def kernel(x_t_feats, x_t_coords, tex_feats, tex_coords, shape_feats,
           shape_coords, t, cond, coords_len_list, W, Wc):
    T, D = x_t_feats.shape
    dc = x_t_coords.shape[1]
    B, DCOND = cond.shape
    blk = T // _GRID
    t2 = t.reshape(1, 1)

    of = pl.pallas_call(
        _body,
        grid=(_GRID,),
        in_specs=[
            pl.BlockSpec((blk, D), lambda i: (i, 0)),
            pl.BlockSpec((blk, D), lambda i: (i, 0)),
            pl.BlockSpec((1, 1), lambda i: (0, 0)),
            pl.BlockSpec((B, DCOND), lambda i: (0, 0)),
            pl.BlockSpec((D, D), lambda i: (0, 0)),
            pl.BlockSpec((DCOND, D), lambda i: (0, 0)),
        ],
        out_specs=pl.BlockSpec((blk, D), lambda i: (i, 0)),
        out_shape=jax.ShapeDtypeStruct((T, D), jnp.float32),
        compiler_params=pltpu.CompilerParams(
            dimension_semantics=("arbitrary",)),
    )(x_t_feats, shape_feats, t2, cond, W, Wc)

    return of, x_t_coords


# speedup vs baseline: 9.3046x; 3.8095x over previous
import jax
import jax.numpy as jnp
from jax.experimental import pallas as pl
from jax.experimental.pallas import tpu as pltpu

_GRID = 8


def _body(x_ref, s_ref, condt_ref, t_ref, wt_ref, wct_ref, of_ref):
    condm = jnp.mean(condt_ref[...], axis=1, keepdims=True)
    cvt = jnp.dot(wct_ref[...], condm,
                  preferred_element_type=jnp.float32)
    tt = t_ref[0, 0]
    xw = jnp.dot(wt_ref[...], x_ref[...],
                 preferred_element_type=jnp.float32)
    sw = jnp.dot(wt_ref[...], s_ref[...],
                 preferred_element_type=jnp.float32)
    of_ref[...] = xw * tt + cvt + jnp.tanh(sw)


def kernel(x_t_feats, x_t_coords, tex_feats, tex_coords, shape_feats,
           shape_coords, t, cond, coords_len_list, W, Wc):
    T, D = x_t_feats.shape
    B, DCOND = cond.shape

    xT = x_t_feats.T
    sT = shape_feats.T
    wT = W.T
    wcT = Wc.T
    condT = cond.T
    t2 = t.reshape(1, 1)
    blk = T // _GRID

    ofT = pl.pallas_call(
        _body,
        grid=(_GRID,),
        in_specs=[
            pl.BlockSpec((D, blk), lambda i: (0, i)),
            pl.BlockSpec((D, blk), lambda i: (0, i)),
            pl.BlockSpec((DCOND, B), lambda i: (0, 0)),
            pl.BlockSpec((1, 1), lambda i: (0, 0)),
            pl.BlockSpec((D, D), lambda i: (0, 0)),
            pl.BlockSpec((D, DCOND), lambda i: (0, 0)),
        ],
        out_specs=pl.BlockSpec((D, blk), lambda i: (0, i)),
        out_shape=jax.ShapeDtypeStruct((D, T), jnp.float32),
        compiler_params=pltpu.CompilerParams(
            dimension_semantics=("arbitrary",)),
    )(xT, sT, condT, t2, wT, wcT)

    return ofT.T, x_t_coords


# speedup vs baseline: 14.0969x; 1.5151x over previous
import jax
import jax.numpy as jnp
from jax import lax
from jax.experimental import pallas as pl
from jax.experimental.pallas import tpu as pltpu

_GRID = 8


def _body(x_ref, s_ref, c_ref, cond_ref, t_ref, w_ref, wct_ref,
          of_ref, oc_ref):
    condm = jnp.mean(cond_ref[...], axis=0, keepdims=True)
    cvt = lax.dot_general(wct_ref[...], condm,
                          (((1,), (1,)), ((), ())),
                          preferred_element_type=jnp.float32)
    tt = t_ref[0, 0]
    xw = lax.dot_general(w_ref[...], x_ref[...],
                         (((0,), (0,)), ((), ())),
                         preferred_element_type=jnp.float32)
    sw = lax.dot_general(w_ref[...], s_ref[...],
                         (((0,), (0,)), ((), ())),
                         preferred_element_type=jnp.float32)
    of_ref[...] = xw * tt + cvt + jnp.tanh(sw)
    oc_ref[...] = c_ref[...]


def kernel(x_t_feats, x_t_coords, tex_feats, tex_coords, shape_feats,
           shape_coords, t, cond, coords_len_list, W, Wc):
    T, D = x_t_feats.shape
    dc = x_t_coords.shape[1]
    B, DCOND = cond.shape

    xT = x_t_feats.T
    sT = shape_feats.T
    cT = x_t_coords.T
    wcT = Wc.T
    t2 = t.reshape(1, 1)
    blk = T // _GRID

    ofT, ocT = pl.pallas_call(
        _body,
        grid=(_GRID,),
        in_specs=[
            pl.BlockSpec((D, blk), lambda i: (0, i)),
            pl.BlockSpec((D, blk), lambda i: (0, i)),
            pl.BlockSpec((dc, blk), lambda i: (0, i)),
            pl.BlockSpec((B, DCOND), lambda i: (0, 0)),
            pl.BlockSpec((1, 1), lambda i: (0, 0)),
            pl.BlockSpec((D, D), lambda i: (0, 0)),
            pl.BlockSpec((D, DCOND), lambda i: (0, 0)),
        ],
        out_specs=[
            pl.BlockSpec((D, blk), lambda i: (0, i)),
            pl.BlockSpec((dc, blk), lambda i: (0, i)),
        ],
        out_shape=[
            jax.ShapeDtypeStruct((D, T), jnp.float32),
            jax.ShapeDtypeStruct((dc, T), jnp.int32),
        ],
        compiler_params=pltpu.CompilerParams(
            dimension_semantics=("arbitrary",)),
    )(xT, sT, cT, cond, t2, W, wcT)

    return ofT.T, ocT.T
